# baseline (speedup 1.0000x reference)
import jax
import jax.numpy as jnp
from jax import lax
from jax.experimental import pallas as pl
from jax.experimental.pallas import tpu as pltpu
from jax.experimental.pallas import tpu_sc as plsc

N_NODES = 10000
N_EDGES = 320000
EDGE_DIM = 16
NODE_DIM = 128

NC = 2
NS = 16
NW = NC * NS
EPW = N_EDGES // NW
BATCH = 100
NB = EPW // BATCH
CHUNK = 2500
N_CHUNKS = EPW // CHUNK
BPC = CHUNK // BATCH
ROWS_PER_TILE = N_NODES // NS
ZROWS = N_NODES // (NS // 2)


def _sc_scatter_body(src3d_hbm, eattr_hbm, out_sums, out_cnts,
                     ebuf, ibuf, ones_v, zb_v, zi_v, sums_sh, cnts_sh):
    c = lax.axis_index("c")
    s = lax.axis_index("s")
    w = c * NS + s

    zb = jnp.zeros((2, 16), jnp.bfloat16)
    zi = jnp.zeros((2, 16), jnp.int16)
    o2 = jnp.ones((2, 16), jnp.int16)

    def fill_zb(i, carry):
        zb_v[pl.ds(2 * i, 2), :] = zb
        return carry

    lax.fori_loop(0, ZROWS // 2, fill_zb, 0)

    def fill_zi(i, carry):
        zi_v[pl.ds(2 * i, 2), :] = zi
        return carry

    lax.fori_loop(0, ZROWS // 2, fill_zi, 0)

    def fill_o(i, carry):
        ones_v[pl.ds(2 * i, 2), :] = o2
        return carry

    lax.fori_loop(0, BATCH // 2, fill_o, 0)

    @pl.when(s < NS // 2)
    def _():
        pltpu.sync_copy(zb_v, sums_sh.at[pl.ds(s * ZROWS, ZROWS)])

    @pl.when(s >= NS // 2)
    def _():
        pltpu.sync_copy(zi_v, cnts_sh.at[pl.ds((s - NS // 2) * ZROWS, ZROWS)])

    pltpu.sync_copy(src3d_hbm.at[w], ibuf)
    plsc.subcore_barrier()

    for k in range(N_CHUNKS):
        ebase = w * EPW + k * CHUNK
        pltpu.sync_copy(eattr_hbm.at[pl.ds(ebase, CHUNK)], ebuf)

        def scat(b, carry):
            idx = ibuf.at[k * BPC + b]
            pltpu.sync_copy(ebuf.at[pl.ds(b * BATCH, BATCH)],
                            sums_sh.at[idx], add=True)
            pltpu.sync_copy(ones_v, cnts_sh.at[idx], add=True)
            return carry

        lax.fori_loop(0, BPC, scat, 0)

    plsc.subcore_barrier()

    row0 = s * ROWS_PER_TILE
    pltpu.sync_copy(sums_sh.at[pl.ds(row0, ROWS_PER_TILE)], out_sums.at[c, s])
    pltpu.sync_copy(cnts_sh.at[pl.ds(row0, ROWS_PER_TILE)], out_cnts.at[c, s])


@jax.jit
def _sc_scatter(src3d, edge_attr_bf16):
    mesh = plsc.VectorSubcoreMesh(core_axis_name="c", subcore_axis_name="s")
    f = pl.kernel(
        _sc_scatter_body,
        out_type=(
            jax.ShapeDtypeStruct((NC, NS, ROWS_PER_TILE, EDGE_DIM), jnp.bfloat16),
            jax.ShapeDtypeStruct((NC, NS, ROWS_PER_TILE, EDGE_DIM), jnp.int16),
        ),
        mesh=mesh,
        compiler_params=pltpu.CompilerParams(use_tc_tiling_on_sc=False),
        scratch_types=[
            pltpu.VMEM((CHUNK, EDGE_DIM), jnp.bfloat16),
            pltpu.VMEM((NB, BATCH), jnp.int32),
            pltpu.VMEM((BATCH, EDGE_DIM), jnp.int16),
            pltpu.VMEM((ZROWS, EDGE_DIM), jnp.bfloat16),
            pltpu.VMEM((ZROWS, EDGE_DIM), jnp.int16),
            pltpu.VMEM_SHARED((N_NODES, EDGE_DIM), jnp.bfloat16),
            pltpu.VMEM_SHARED((N_NODES, EDGE_DIM), jnp.int16),
        ],
    )
    return f(src3d, edge_attr_bf16)


def _tc_mlp_body(x_ref, sums_ref, cnts_ref, w1a_ref, w1b_ref, b1_ref,
                 w2_ref, b2_ref, w3_ref, b3_ref, wo_ref, bo_ref,
                 g_ref, bt_ref, out_ref):
    sums = (sums_ref[0].astype(jnp.float32) + sums_ref[1].astype(jnp.float32))
    cnt = (cnts_ref[0, :, 0:1] + cnts_ref[1, :, 0:1]).astype(jnp.float32)
    agg = sums / jnp.maximum(cnt, 1.0)

    g = g_ref[...]
    bt = bt_ref[...]

    h = (jnp.dot(x_ref[...], w1a_ref[...], preferred_element_type=jnp.float32)
         + jnp.dot(agg, w1b_ref[...], preferred_element_type=jnp.float32)
         + b1_ref[...])

    for w_ref, b_ref in ((w2_ref, b2_ref), (w3_ref, b3_ref), (None, None)):
        h = jnp.maximum(h, 0.0)
        mu = jnp.mean(h, axis=0, keepdims=True)
        d = h - mu
        var = jnp.mean(d * d, axis=0, keepdims=True)
        h = g * d / jnp.sqrt(var + 1e-5) + bt
        if w_ref is not None:
            h = jnp.dot(h, w_ref[...], preferred_element_type=jnp.float32) + b_ref[...]

    out_ref[...] = (jnp.dot(h, wo_ref[...], preferred_element_type=jnp.float32)
                    + bo_ref[...])


@jax.jit
def _tc_mlp(x, sums, cnts, w1a, w1b, b1, w2, b2, w3, b3, wo, bo, g, bt):
    return pl.pallas_call(
        _tc_mlp_body,
        out_shape=jax.ShapeDtypeStruct((N_NODES, 64), jnp.float32),
    )(x, sums, cnts, w1a, w1b, b1, w2, b2, w3, b3, wo, bo, g, bt)


def kernel(x, edge_index, edge_attr, W1, b1, W2, b2, W3, b3, Wout, bout,
           gamma, beta):
    src = edge_index[0].astype(jnp.int32)
    src3d = src.reshape(NW, NB, BATCH)
    sums, cnts = _sc_scatter(src3d, edge_attr.astype(jnp.bfloat16))
    sums = sums.reshape(NC, N_NODES, EDGE_DIM)
    cnts = cnts.reshape(NC, N_NODES, EDGE_DIM)
    r = lambda v: v.reshape(1, -1)
    return _tc_mlp(x, sums, cnts, W1[:NODE_DIM], W1[NODE_DIM:], r(b1),
                   W2, r(b2), W3, r(b3), Wout, r(bout), r(gamma), r(beta))


# speedup vs baseline: 1.0768x; 1.0768x over previous
import jax
import jax.numpy as jnp
from jax import lax
from jax.experimental import pallas as pl
from jax.experimental.pallas import tpu as pltpu
from jax.experimental.pallas import tpu_sc as plsc

N_NODES = 10000
N_EDGES = 320000
EDGE_DIM = 16
NODE_DIM = 128

NC = 2
NS = 16
NW = NC * NS
EPW = N_EDGES // NW
BATCH = 80
NB = EPW // BATCH
CHUNK = 2000
N_CHUNKS = EPW // CHUNK
BPC = CHUNK // BATCH
ROWS_PER_TILE = N_NODES // NS


def _sc_scatter_body(src_hbm, eattr_hbm, out_sums, out_cnts,
                     ebuf, ibuf1, ibuf2, ones_v, z2d, sums_sh, cnts_sh):
    c = lax.axis_index("c")
    s = lax.axis_index("s")
    w = c * NS + s

    z16 = jnp.zeros((16,), jnp.float32)
    o16 = jnp.ones((16,), jnp.float32)

    def fill_z(i, carry):
        z2d[i, :] = z16
        return carry

    lax.fori_loop(0, ROWS_PER_TILE, fill_z, 0)

    def fill_o(i, carry):
        ones_v[i, :] = o16
        return carry

    lax.fori_loop(0, BATCH, fill_o, 0)

    pltpu.sync_copy(src_hbm.at[pl.ds(w * EPW, EPW)], ibuf1)

    def repack_i(i, carry):
        v = ibuf1[pl.ds(i * 16, 16)]
        ibuf2[i // (BATCH // 16), pl.ds((i % (BATCH // 16)) * 16, 16)] = v
        return carry

    lax.fori_loop(0, EPW // 16, repack_i, 0)

    pltpu.sync_copy(z2d, sums_sh.at[pl.ds(s * ROWS_PER_TILE, ROWS_PER_TILE)])
    pltpu.sync_copy(z2d, cnts_sh.at[pl.ds(s * ROWS_PER_TILE, ROWS_PER_TILE)])
    plsc.subcore_barrier()

    for k in range(N_CHUNKS):
        ebase = w * EPW + k * CHUNK
        pltpu.sync_copy(eattr_hbm.at[pl.ds(ebase, CHUNK)], ebuf)

        def scat(b, carry):
            idx = ibuf2.at[k * BPC + b]
            pltpu.sync_copy(ebuf.at[pl.ds(b * BATCH, BATCH)],
                            sums_sh.at[idx], add=True)
            pltpu.sync_copy(ones_v, cnts_sh.at[idx], add=True)
            return carry

        lax.fori_loop(0, BPC, scat, 0)

    plsc.subcore_barrier()

    row0 = s * ROWS_PER_TILE
    pltpu.sync_copy(sums_sh.at[pl.ds(row0, ROWS_PER_TILE)], out_sums.at[c, s])
    pltpu.sync_copy(cnts_sh.at[pl.ds(row0, ROWS_PER_TILE)], out_cnts.at[c, s])


@jax.jit
def _sc_scatter(src, edge_attr):
    mesh = plsc.VectorSubcoreMesh(core_axis_name="c", subcore_axis_name="s")
    f = pl.kernel(
        _sc_scatter_body,
        out_type=(
            jax.ShapeDtypeStruct((NC, NS, ROWS_PER_TILE, EDGE_DIM), jnp.float32),
            jax.ShapeDtypeStruct((NC, NS, ROWS_PER_TILE, EDGE_DIM), jnp.float32),
        ),
        mesh=mesh,
        compiler_params=pltpu.CompilerParams(use_tc_tiling_on_sc=False),
        scratch_types=[
            pltpu.VMEM((CHUNK, EDGE_DIM), jnp.float32),
            pltpu.VMEM((EPW,), jnp.int32),
            pltpu.VMEM((NB, BATCH), jnp.int32),
            pltpu.VMEM((BATCH, EDGE_DIM), jnp.float32),
            pltpu.VMEM((ROWS_PER_TILE, EDGE_DIM), jnp.float32),
            pltpu.VMEM_SHARED((N_NODES, EDGE_DIM), jnp.float32),
            pltpu.VMEM_SHARED((N_NODES, EDGE_DIM), jnp.float32),
        ],
    )
    return f(src, edge_attr)


def _tc_mlp_body(x_ref, sums_ref, cnts_ref, w1a_ref, w1b_ref, b1_ref,
                 w2_ref, b2_ref, w3_ref, b3_ref, wo_ref, bo_ref,
                 g_ref, bt_ref, out_ref):
    sums = sums_ref[0] + sums_ref[1]
    cnt = cnts_ref[0, :, 0:1] + cnts_ref[1, :, 0:1]
    agg = sums / jnp.maximum(cnt, 1.0)

    g = g_ref[...]
    bt = bt_ref[...]

    h = (jnp.dot(x_ref[...], w1a_ref[...], preferred_element_type=jnp.float32)
         + jnp.dot(agg, w1b_ref[...], preferred_element_type=jnp.float32)
         + b1_ref[...])

    for w_ref, b_ref in ((w2_ref, b2_ref), (w3_ref, b3_ref), (None, None)):
        h = jnp.maximum(h, 0.0)
        mu = jnp.mean(h, axis=0, keepdims=True)
        d = h - mu
        var = jnp.mean(d * d, axis=0, keepdims=True)
        h = g * d / jnp.sqrt(var + 1e-5) + bt
        if w_ref is not None:
            h = jnp.dot(h, w_ref[...], preferred_element_type=jnp.float32) + b_ref[...]

    out_ref[...] = (jnp.dot(h, wo_ref[...], preferred_element_type=jnp.float32)
                    + bo_ref[...])


@jax.jit
def _tc_mlp(x, sums, cnts, w1a, w1b, b1, w2, b2, w3, b3, wo, bo, g, bt):
    return pl.pallas_call(
        _tc_mlp_body,
        out_shape=jax.ShapeDtypeStruct((N_NODES, 64), jnp.float32),
    )(x, sums, cnts, w1a, w1b, b1, w2, b2, w3, b3, wo, bo, g, bt)


def kernel(x, edge_index, edge_attr, W1, b1, W2, b2, W3, b3, Wout, bout,
           gamma, beta):
    src = edge_index[0].astype(jnp.int32)
    sums, cnts = _sc_scatter(src, edge_attr)
    sums = sums.reshape(NC, N_NODES, EDGE_DIM)
    cnts = cnts.reshape(NC, N_NODES, EDGE_DIM)
    r = lambda v: v.reshape(1, -1)
    return _tc_mlp(x, sums, cnts, W1[:NODE_DIM], W1[NODE_DIM:], r(b1),
                   W2, r(b2), W3, r(b3), Wout, r(bout), r(gamma), r(beta))
